# baseline (speedup 1.0000x reference)
import functools

import jax
import jax.numpy as jnp
from jax import lax
from jax.experimental import pallas as pl
from jax.experimental.pallas import tpu as pltpu
from jax.experimental.pallas import tpu_sc as plsc

VOCAB = 1000000
DIM = 64
B = 4096
L = 50

_info = plsc.get_sparse_core_info()
NC, NS = _info.num_cores, _info.num_subcores
NW = NC * NS
TOTAL = B * L
CHUNK = 128
NCHUNK = TOTAL // (NW * CHUNK)

_mesh = plsc.VectorSubcoreMesh(core_axis_name="c", subcore_axis_name="s")


@functools.partial(
    pl.kernel,
    mesh=_mesh,
    out_type=jax.ShapeDtypeStruct((NW, NCHUNK, CHUNK, DIM), jnp.float32),
    scratch_types=[
        pltpu.VMEM((NCHUNK, CHUNK), jnp.int32),
        pltpu.VMEM((CHUNK, DIM), jnp.float32),
        pltpu.SemaphoreType.DMA,
    ],
    compiler_params=pltpu.CompilerParams(use_tc_tiling_on_sc=False),
)
def _gather(table_hbm, idx_hbm, out_hbm, idx_v, rows_v, sem):
    wid = lax.axis_index("s") * NC + lax.axis_index("c")
    pltpu.sync_copy(idx_hbm.at[wid], idx_v)

    def step(j, carry):
        pltpu.async_copy(table_hbm.at[idx_v.at[j]], rows_v, sem).wait()
        pltpu.sync_copy(rows_v, out_hbm.at[wid, j])
        return carry

    lax.fori_loop(0, NCHUNK, step, 0)


def kernel(input_, weight):
    idx = input_.reshape(NW, NCHUNK, CHUNK).astype(jnp.int32)
    out = _gather(weight, idx)
    return out.reshape(B, L, DIM)


# speedup vs baseline: 1.0437x; 1.0437x over previous
import functools

import jax
import jax.numpy as jnp
from jax import lax
from jax.experimental import pallas as pl
from jax.experimental.pallas import tpu as pltpu
from jax.experimental.pallas import tpu_sc as plsc

VOCAB = 1000000
DIM = 64
B = 4096
L = 50

_info = plsc.get_sparse_core_info()
NC, NS = _info.num_cores, _info.num_subcores
NW = NC * NS
TOTAL = B * L
CHUNK = 128
NCHUNK = TOTAL // (NW * CHUNK)
GPB = 5
NG = NCHUNK // GPB

_mesh = plsc.VectorSubcoreMesh(core_axis_name="c", subcore_axis_name="s")


@functools.partial(
    pl.kernel,
    mesh=_mesh,
    out_type=jax.ShapeDtypeStruct((NW, NCHUNK, CHUNK, DIM), jnp.float32),
    scratch_types=[
        pltpu.VMEM((NCHUNK, CHUNK), jnp.int32),
        pltpu.VMEM((2, GPB, CHUNK, DIM), jnp.float32),
        pltpu.SemaphoreType.DMA((2,)),
        pltpu.SemaphoreType.DMA((2,)),
    ],
    compiler_params=pltpu.CompilerParams(use_tc_tiling_on_sc=False),
)
def _gather(table_hbm, idx_hbm, out_hbm, idx_v, rows_v, sem_g, sem_w):
    wid = lax.axis_index("s") * NC + lax.axis_index("c")
    pltpu.sync_copy(idx_hbm.at[wid], idx_v)

    def fire_group(g, bb):
        for k in range(GPB):
            pltpu.make_async_copy(
                table_hbm.at[idx_v.at[g * GPB + k]],
                rows_v.at[bb, k],
                sem_g.at[bb],
            ).start()

    def drain_group(g, bb):
        for k in range(GPB):
            pltpu.make_async_copy(
                table_hbm.at[idx_v.at[g * GPB + k]],
                rows_v.at[bb, k],
                sem_g.at[bb],
            ).wait()

    def write_group(g, bb):
        return pltpu.make_async_copy(
            rows_v.at[bb],
            out_hbm.at[wid, pl.ds(g * GPB, GPB)],
            sem_w.at[bb],
        )

    fire_group(0, 0)

    def step(g, carry):
        bb = lax.rem(g, 2)
        pb = 1 - bb

        @pl.when(g >= 2)
        def _():
            write_group(g - 2, bb).wait()

        fire_group(g, bb)
        drain_group(g - 1, pb)
        write_group(g - 1, pb).start()
        return carry

    lax.fori_loop(1, NG, step, 0)

    last = NG - 1
    lb = last % 2
    write_group(last - 1, 1 - lb).wait()
    drain_group(last, lb)
    wlast = write_group(last, lb)
    wlast.start()
    wlast.wait()


def kernel(input_, weight):
    idx = input_.reshape(NW, NCHUNK, CHUNK).astype(jnp.int32)
    out = _gather(weight, idx)
    return out.reshape(B, L, DIM)


# speedup vs baseline: 1.0496x; 1.0056x over previous
import functools

import jax
import jax.numpy as jnp
from jax import lax
from jax.experimental import pallas as pl
from jax.experimental.pallas import tpu as pltpu
from jax.experimental.pallas import tpu_sc as plsc

VOCAB = 1000000
DIM = 64
B = 4096
L = 50

_info = plsc.get_sparse_core_info()
NC, NS = _info.num_cores, _info.num_subcores
NW = NC * NS
CHUNK = 128
NCHUNK = L
BPW = B // NW

_mesh = plsc.VectorSubcoreMesh(core_axis_name="c", subcore_axis_name="s")


@functools.partial(
    pl.kernel,
    mesh=_mesh,
    out_type=jax.ShapeDtypeStruct((B * L, 2 * DIM), jnp.float32),
    scratch_types=[
        pltpu.VMEM((NCHUNK, CHUNK), jnp.int32),
        pltpu.VMEM((2, CHUNK, 2 * DIM), jnp.float32),
        pltpu.SemaphoreType.DMA((2,)),
        pltpu.SemaphoreType.DMA((2,)),
    ],
    compiler_params=pltpu.CompilerParams(use_tc_tiling_on_sc=True),
)
def _gather(table_hbm, idx_hbm, out_hbm, idx_v, rows_v, sem_g, sem_w):
    wid = lax.axis_index("s") * NC + lax.axis_index("c")
    pltpu.sync_copy(idx_hbm.at[wid], idx_v)

    def gather_chunk(g, bb):
        return pltpu.make_async_copy(
            table_hbm.at[idx_v.at[g]], rows_v.at[bb], sem_g.at[bb]
        )

    def write_chunk(g, bb):
        return pltpu.make_async_copy(
            rows_v.at[bb],
            out_hbm.at[pl.ds(wid * NCHUNK * CHUNK + g * CHUNK, CHUNK)],
            sem_w.at[bb],
        )

    gather_chunk(0, 0).start()

    def step(g, carry):
        bb = lax.rem(g, 2)
        pb = 1 - bb

        @pl.when(g >= 2)
        def _():
            write_chunk(g - 2, bb).wait()

        gather_chunk(g, bb).start()
        gather_chunk(g - 1, pb).wait()
        write_chunk(g - 1, pb).start()
        return carry

    lax.fori_loop(1, NCHUNK, step, 0)

    last = NCHUNK - 1
    lb = last % 2
    write_chunk(last - 1, 1 - lb).wait()
    gather_chunk(last, lb).wait()
    wlast = write_chunk(last, lb)
    wlast.start()
    wlast.wait()


def kernel(input_, weight):
    wide = jnp.pad(weight, ((0, 0), (0, DIM)))
    idx3 = input_.reshape(NW, NCHUNK, CHUNK).astype(jnp.int32)
    out = _gather(wide, idx3)
    return out[:, :DIM].reshape(B, L, DIM)
